# Initial kernel scaffold; baseline (speedup 1.0000x reference)
import jax
import jax.numpy as jnp
from jax.experimental import pallas as pl


def kernel(x, edge_index, batch, W1, b1, W2, b2, Wl, bl):
    raise NotImplementedError("write your pallas kernel here")


# speedup vs baseline: 12.5184x; 12.5184x over previous
import functools

import jax
import jax.numpy as jnp
from jax import lax
from jax.experimental import pallas as pl
from jax.experimental.pallas import tpu as pltpu
from jax.experimental.pallas import tpu_sc as plsc

N_NODES = 10000
N_EDGES = 320000
FEATS = 128
NUM_GRAPHS = 64
NUM_CLASSES = 10

NC = 2
NS = 16

RCH = 624
TAIL0 = RCH * NS
TAIL = N_NODES - TAIL0
ZT = 156

W_EDGE = 80

TCB = 2000
TCG = N_NODES // TCB

_HIGH = jax.lax.Precision.HIGHEST


def _mesh():
    return plsc.VectorSubcoreMesh(core_axis_name="c", subcore_axis_name="s")


def _fill(tile, rows, width, value):
    v16 = jnp.full((16,), value, jnp.float32)

    @pl.loop(0, rows)
    def _(i):
        @pl.loop(0, width, step=16)
        def _(j):
            tile[i, pl.ds(j, 16)] = v16


def _clear_and_readout(sid, acc_sp, ztile, out_ref, phase):
    r0 = sid * RCH
    if phase == 0:
        @pl.loop(0, RCH // ZT)
        def _(i):
            pltpu.sync_copy(ztile, acc_sp.at[pl.ds(r0 + i * ZT, ZT)])

        @pl.when(sid == 0)
        def _():
            pltpu.sync_copy(ztile.at[pl.ds(0, TAIL)], acc_sp.at[pl.ds(TAIL0, TAIL)])
    else:
        pltpu.sync_copy(acc_sp.at[pl.ds(r0, RCH)], out_ref.at[pl.ds(r0, RCH)])

        @pl.when(sid == 0)
        def _():
            pltpu.sync_copy(acc_sp.at[pl.ds(TAIL0, TAIL)],
                            out_ref.at[pl.ds(TAIL0, TAIL)])


def _deg_sc(dst):
    @functools.partial(
        pl.kernel,
        out_type=jax.ShapeDtypeStruct((NC, N_NODES, FEATS), jnp.float32),
        mesh=_mesh(),
        scratch_types=[
            pltpu.VMEM((W_EDGE,), jnp.int32),
            pltpu.VMEM((W_EDGE, FEATS), jnp.float32),
            pltpu.VMEM((ZT, FEATS), jnp.float32),
            pltpu.VMEM_SHARED((N_NODES, FEATS), jnp.float32),
        ],
    )
    def k(dst_hbm, out_hbm, didx, ones_v, ztile, deg_sp):
        cid = lax.axis_index("c")
        sid = lax.axis_index("s")
        _fill(ones_v, W_EDGE, FEATS, 1.0)
        _fill(ztile, ZT, FEATS, 0.0)
        _clear_and_readout(sid, deg_sp, ztile, None, 0)
        plsc.subcore_barrier()

        e0 = (cid * NS + sid) * (N_EDGES // (NC * NS))

        @pl.loop(0, (N_EDGES // (NC * NS)) // W_EDGE)
        def _(w):
            pltpu.sync_copy(dst_hbm.at[pl.ds(e0 + w * W_EDGE, W_EDGE)], didx)
            pltpu.sync_copy(ones_v, deg_sp.at[didx], add=True)

        plsc.subcore_barrier()
        _clear_and_readout(sid, deg_sp, ztile, out_hbm.at[cid], 1)

    return k(dst)


def _spmm_sc(y, src, dst):
    eps = N_EDGES // (NC * NS)

    @functools.partial(
        pl.kernel,
        out_type=jax.ShapeDtypeStruct((NC, N_NODES, FEATS), jnp.float32),
        mesh=_mesh(),
        scratch_types=[
            pltpu.VMEM((W_EDGE,), jnp.int32),
            pltpu.VMEM((W_EDGE,), jnp.int32),
            pltpu.VMEM((W_EDGE, FEATS), jnp.float32),
            pltpu.VMEM((ZT, FEATS), jnp.float32),
            pltpu.VMEM_SHARED((N_NODES, FEATS), jnp.float32),
        ],
    )
    def k(y_hbm, src_hbm, dst_hbm, out_hbm, sidx, didx, rows, ztile, z_sp):
        cid = lax.axis_index("c")
        sid = lax.axis_index("s")

        _fill(ztile, ZT, FEATS, 0.0)
        _clear_and_readout(sid, z_sp, ztile, None, 0)
        plsc.subcore_barrier()

        e0 = (cid * NS + sid) * eps

        @pl.loop(0, eps // W_EDGE)
        def _(w):
            base = e0 + w * W_EDGE
            pltpu.sync_copy(src_hbm.at[pl.ds(base, W_EDGE)], sidx)
            pltpu.sync_copy(dst_hbm.at[pl.ds(base, W_EDGE)], didx)
            pltpu.sync_copy(y_hbm.at[sidx], rows)
            pltpu.sync_copy(rows, z_sp.at[didx], add=True)

        plsc.subcore_barrier()
        _clear_and_readout(sid, z_sp, ztile, out_hbm.at[cid], 1)

    return k(y, src, dst)


def _tc_dinv(deg2d):
    def body(deg_ref, o_ref):
        d = deg_ref[...]
        deg = (d[0, :, 0] + d[1, :, 0]).reshape(TCB, 1) + 1.0
        o_ref[...] = jnp.broadcast_to(1.0 / jnp.sqrt(deg), (TCB, 8))

    return pl.pallas_call(
        body,
        grid=(TCG,),
        in_specs=[_zrows_spec()],
        out_specs=_rows_spec(8),
        out_shape=jax.ShapeDtypeStruct((N_NODES, 8), jnp.float32),
    )(deg2d)


def _dinv_block(dinv_ref):
    return dinv_ref[...][:, 0:1]


def _rows_spec(width=FEATS):
    return pl.BlockSpec((TCB, width), lambda i: (i, 0))


def _zrows_spec(width=FEATS):
    return pl.BlockSpec((NC, TCB, width), lambda i: (0, i, 0))


def _full(shape):
    n = len(shape)
    return pl.BlockSpec(shape, lambda i, _n=n: (0,) * _n)


def _tc_pre(x, W1, dinv8):
    def body(x_ref, w_ref, dinv_ref, y_ref):
        dinv = _dinv_block(dinv_ref)
        xw = lax.dot_general(x_ref[...], w_ref[...], (((1,), (0,)), ((), ())),
                             precision=_HIGH)
        y_ref[...] = xw * dinv

    return pl.pallas_call(
        body,
        grid=(TCG,),
        in_specs=[_rows_spec(), _full((FEATS, FEATS)), _rows_spec(8)],
        out_specs=_rows_spec(),
        out_shape=jax.ShapeDtypeStruct((N_NODES, FEATS), jnp.float32),
    )(x, W1, dinv8)


def _tc_mid(z1, y1, dinv8, b1, W2):
    def body(z_ref, y_ref, dinv_ref, b_ref, w_ref, o_ref):
        dinv = _dinv_block(dinv_ref)
        z = z_ref[0] + z_ref[1] + y_ref[...]
        h = jnp.maximum(z * dinv + b_ref[...], 0.0)
        o_ref[...] = lax.dot_general(h, w_ref[...], (((1,), (0,)), ((), ())),
                                     precision=_HIGH) * dinv

    return pl.pallas_call(
        body,
        grid=(TCG,),
        in_specs=[_zrows_spec(), _rows_spec(), _rows_spec(8),
                  _full((1, FEATS)), _full((FEATS, FEATS))],
        out_specs=_rows_spec(),
        out_shape=jax.ShapeDtypeStruct((N_NODES, FEATS), jnp.float32),
    )(z1, y1, dinv8, b1, W2)


def _tc_post(z2, y2, dinv8, b2, batchc, Wl, bl):
    def body(z_ref, y_ref, dinv_ref, b_ref, batch_ref, wl_ref, bl_ref, o_ref,
             sum_s, cnt_s):
        i = pl.program_id(0)

        @pl.when(i == 0)
        def _():
            sum_s[...] = jnp.zeros((NUM_GRAPHS, FEATS), jnp.float32)
            cnt_s[...] = jnp.zeros((NUM_GRAPHS, FEATS), jnp.float32)

        dinv = _dinv_block(dinv_ref)
        z = z_ref[0] + z_ref[1] + y_ref[...]
        h = jnp.maximum(z * dinv + b_ref[...], 0.0)
        gids = lax.broadcasted_iota(jnp.int32, (NUM_GRAPHS, TCB), 0)
        onehot = (gids == batch_ref[...].reshape(1, TCB)).astype(jnp.float32)
        sum_s[...] += lax.dot_general(onehot, h, (((1,), (0,)), ((), ())),
                                      precision=_HIGH)
        cnt_s[...] += jnp.broadcast_to(
            jnp.sum(onehot, axis=1, keepdims=True), (NUM_GRAPHS, FEATS))

        @pl.when(i == TCG - 1)
        def _():
            pooled = sum_s[...] / jnp.maximum(cnt_s[...], 1.0)
            o_ref[...] = lax.dot_general(
                pooled, wl_ref[...], (((1,), (0,)), ((), ())),
                precision=_HIGH) + bl_ref[...]

    return pl.pallas_call(
        body,
        grid=(TCG,),
        in_specs=[_zrows_spec(), _rows_spec(), _rows_spec(8),
                  _full((1, FEATS)),
                  pl.BlockSpec((1, 1, TCB), lambda i: (i, 0, 0)),
                  _full((FEATS, NUM_CLASSES)), _full((1, NUM_CLASSES))],
        out_specs=pl.BlockSpec((NUM_GRAPHS, NUM_CLASSES), lambda i: (0, 0)),
        out_shape=jax.ShapeDtypeStruct((NUM_GRAPHS, NUM_CLASSES), jnp.float32),
        scratch_shapes=[pltpu.VMEM((NUM_GRAPHS, FEATS), jnp.float32),
                        pltpu.VMEM((NUM_GRAPHS, FEATS), jnp.float32)],
    )(z2, y2, dinv8, b2, batchc, Wl, bl)


def kernel(x, edge_index, batch, W1, b1, W2, b2, Wl, bl):
    src = edge_index[0].astype(jnp.int32)
    dst = edge_index[1].astype(jnp.int32)
    batchc = batch.astype(jnp.int32).reshape(TCG, 1, TCB)

    deg2d = _deg_sc(dst)
    dinv8 = _tc_dinv(deg2d)
    y1 = _tc_pre(x, W1, dinv8)
    z1 = _spmm_sc(y1, src, dst)
    y2 = _tc_mid(z1, y1, dinv8, b1.reshape(1, FEATS), W2)
    z2 = _spmm_sc(y2, src, dst)
    return _tc_post(z2, y2, dinv8, b2.reshape(1, FEATS), batchc,
                    Wl, bl.reshape(1, NUM_CLASSES))


# speedup vs baseline: 22.4638x; 1.7945x over previous
import functools

import jax
import jax.numpy as jnp
from jax import lax
from jax.experimental import pallas as pl
from jax.experimental.pallas import tpu as pltpu
from jax.experimental.pallas import tpu_sc as plsc

N_NODES = 10000
N_EDGES = 320000
FEATS = 128
NUM_GRAPHS = 64
NUM_CLASSES = 10

NC = 2
NS = 16

RCH = 624
TAIL0 = RCH * NS
TAIL = N_NODES - TAIL0
ZT = 156

W_EDGE = 40
KDEPTH = 5

TCB = 2000
TCG = N_NODES // TCB

_HIGH = jax.lax.Precision.HIGHEST


def _mesh():
    return plsc.VectorSubcoreMesh(core_axis_name="c", subcore_axis_name="s")


def _fill(tile, rows, width, value):
    v16 = jnp.full((16,), value, jnp.float32)

    @pl.loop(0, rows)
    def _(i):
        @pl.loop(0, width, step=16)
        def _(j):
            tile[i, pl.ds(j, 16)] = v16


def _clear_and_readout(sid, acc_sp, ztile, out_ref, phase):
    r0 = sid * RCH
    if phase == 0:
        @pl.loop(0, RCH // ZT)
        def _(i):
            pltpu.sync_copy(ztile, acc_sp.at[pl.ds(r0 + i * ZT, ZT)])

        @pl.when(sid == 0)
        def _():
            pltpu.sync_copy(ztile.at[pl.ds(0, TAIL)], acc_sp.at[pl.ds(TAIL0, TAIL)])
    else:
        pltpu.sync_copy(acc_sp.at[pl.ds(r0, RCH)], out_ref.at[pl.ds(r0, RCH)])

        @pl.when(sid == 0)
        def _():
            pltpu.sync_copy(acc_sp.at[pl.ds(TAIL0, TAIL)],
                            out_ref.at[pl.ds(TAIL0, TAIL)])


def _deg_sc(dst):
    @functools.partial(
        pl.kernel,
        out_type=jax.ShapeDtypeStruct((NC, N_NODES, FEATS), jnp.float32),
        mesh=_mesh(),
        scratch_types=(
            [pltpu.VMEM((W_EDGE,), jnp.int32)] * KDEPTH
            + [pltpu.VMEM((W_EDGE, FEATS), jnp.float32),
               pltpu.VMEM((ZT, FEATS), jnp.float32),
               pltpu.VMEM_SHARED((N_NODES, FEATS), jnp.float32),
               pltpu.SemaphoreType.DMA, pltpu.SemaphoreType.DMA]
        ),
    )
    def k(dst_hbm, out_hbm, *scr):
        didx = scr[:KDEPTH]
        ones_v, ztile, deg_sp, sem_i, sem_s = scr[KDEPTH:]
        cid = lax.axis_index("c")
        sid = lax.axis_index("s")
        _fill(ones_v, W_EDGE, FEATS, 1.0)
        _fill(ztile, ZT, FEATS, 0.0)
        _clear_and_readout(sid, deg_sp, ztile, None, 0)
        plsc.subcore_barrier()

        e0 = (cid * NS + sid) * (N_EDGES // (NC * NS))

        @pl.loop(0, (N_EDGES // (NC * NS)) // (W_EDGE * KDEPTH))
        def _(g):
            base = e0 + g * (W_EDGE * KDEPTH)

            @pl.when(g > 0)
            def _():
                for j in range(KDEPTH):
                    pltpu.make_async_copy(ones_v, deg_sp.at[didx[j]],
                                          sem_s).wait()

            for j in range(KDEPTH):
                pltpu.async_copy(dst_hbm.at[pl.ds(base + j * W_EDGE, W_EDGE)],
                                 didx[j], sem_i)
            for j in range(KDEPTH):
                pltpu.make_async_copy(
                    dst_hbm.at[pl.ds(base + j * W_EDGE, W_EDGE)], didx[j],
                    sem_i).wait()
            for j in range(KDEPTH):
                pltpu.async_copy(ones_v, deg_sp.at[didx[j]], sem_s, add=True)

        for j in range(KDEPTH):
            pltpu.make_async_copy(ones_v, deg_sp.at[didx[j]], sem_s).wait()

        plsc.subcore_barrier()
        _clear_and_readout(sid, deg_sp, ztile, out_hbm.at[cid], 1)

    return k(dst)


def _spmm_sc(y, src, dst):
    eps = N_EDGES // (NC * NS)

    @functools.partial(
        pl.kernel,
        out_type=jax.ShapeDtypeStruct((NC, N_NODES, FEATS), jnp.float32),
        mesh=_mesh(),
        scratch_types=(
            [pltpu.VMEM((W_EDGE,), jnp.int32)] * (2 * KDEPTH)
            + [pltpu.VMEM((W_EDGE, FEATS), jnp.float32)] * KDEPTH
            + [pltpu.VMEM((ZT, FEATS), jnp.float32),
               pltpu.VMEM_SHARED((N_NODES, FEATS), jnp.float32),
               pltpu.SemaphoreType.DMA, pltpu.SemaphoreType.DMA,
               pltpu.SemaphoreType.DMA]
        ),
    )
    def k(y_hbm, src_hbm, dst_hbm, out_hbm, *scr):
        sidx = scr[:KDEPTH]
        didx = scr[KDEPTH:2 * KDEPTH]
        rows = scr[2 * KDEPTH:3 * KDEPTH]
        ztile, z_sp, sem_i, sem_g, sem_s = scr[3 * KDEPTH:]
        cid = lax.axis_index("c")
        sid = lax.axis_index("s")

        _fill(ztile, ZT, FEATS, 0.0)
        _clear_and_readout(sid, z_sp, ztile, None, 0)
        plsc.subcore_barrier()

        e0 = (cid * NS + sid) * eps

        @pl.loop(0, eps // (W_EDGE * KDEPTH))
        def _(g):
            base = e0 + g * (W_EDGE * KDEPTH)

            @pl.when(g > 0)
            def _():
                for j in range(KDEPTH):
                    pltpu.make_async_copy(rows[j], z_sp.at[didx[j]],
                                          sem_s).wait()

            for j in range(KDEPTH):
                pltpu.async_copy(src_hbm.at[pl.ds(base + j * W_EDGE, W_EDGE)],
                                 sidx[j], sem_i)
                pltpu.async_copy(dst_hbm.at[pl.ds(base + j * W_EDGE, W_EDGE)],
                                 didx[j], sem_i)
            for j in range(KDEPTH):
                pltpu.make_async_copy(
                    src_hbm.at[pl.ds(base + j * W_EDGE, W_EDGE)], sidx[j],
                    sem_i).wait()
                pltpu.make_async_copy(
                    dst_hbm.at[pl.ds(base + j * W_EDGE, W_EDGE)], didx[j],
                    sem_i).wait()
            for j in range(KDEPTH):
                pltpu.async_copy(y_hbm.at[sidx[j]], rows[j], sem_g)
            for j in range(KDEPTH):
                pltpu.make_async_copy(y_hbm.at[sidx[j]], rows[j], sem_g).wait()
                pltpu.async_copy(rows[j], z_sp.at[didx[j]], sem_s, add=True)

        for j in range(KDEPTH):
            pltpu.make_async_copy(rows[j], z_sp.at[didx[j]], sem_s).wait()

        plsc.subcore_barrier()
        _clear_and_readout(sid, z_sp, ztile, out_hbm.at[cid], 1)

    return k(y, src, dst)


def _tc_dinv(deg2d):
    def body(deg_ref, o_ref):
        d = deg_ref[...]
        deg = (d[0, :, 0] + d[1, :, 0]).reshape(TCB, 1) + 1.0
        o_ref[...] = jnp.broadcast_to(1.0 / jnp.sqrt(deg), (TCB, 8))

    return pl.pallas_call(
        body,
        grid=(TCG,),
        in_specs=[_zrows_spec()],
        out_specs=_rows_spec(8),
        out_shape=jax.ShapeDtypeStruct((N_NODES, 8), jnp.float32),
    )(deg2d)


def _dinv_block(dinv_ref):
    return dinv_ref[...][:, 0:1]


def _rows_spec(width=FEATS):
    return pl.BlockSpec((TCB, width), lambda i: (i, 0))


def _zrows_spec(width=FEATS):
    return pl.BlockSpec((NC, TCB, width), lambda i: (0, i, 0))


def _full(shape):
    n = len(shape)
    return pl.BlockSpec(shape, lambda i, _n=n: (0,) * _n)


def _tc_pre(x, W1, dinv8):
    def body(x_ref, w_ref, dinv_ref, y_ref):
        dinv = _dinv_block(dinv_ref)
        xw = lax.dot_general(x_ref[...], w_ref[...], (((1,), (0,)), ((), ())),
                             precision=_HIGH)
        y_ref[...] = xw * dinv

    return pl.pallas_call(
        body,
        grid=(TCG,),
        in_specs=[_rows_spec(), _full((FEATS, FEATS)), _rows_spec(8)],
        out_specs=_rows_spec(),
        out_shape=jax.ShapeDtypeStruct((N_NODES, FEATS), jnp.float32),
    )(x, W1, dinv8)


def _tc_mid(z1, y1, dinv8, b1, W2):
    def body(z_ref, y_ref, dinv_ref, b_ref, w_ref, o_ref):
        dinv = _dinv_block(dinv_ref)
        z = z_ref[0] + z_ref[1] + y_ref[...]
        h = jnp.maximum(z * dinv + b_ref[...], 0.0)
        o_ref[...] = lax.dot_general(h, w_ref[...], (((1,), (0,)), ((), ())),
                                     precision=_HIGH) * dinv

    return pl.pallas_call(
        body,
        grid=(TCG,),
        in_specs=[_zrows_spec(), _rows_spec(), _rows_spec(8),
                  _full((1, FEATS)), _full((FEATS, FEATS))],
        out_specs=_rows_spec(),
        out_shape=jax.ShapeDtypeStruct((N_NODES, FEATS), jnp.float32),
    )(z1, y1, dinv8, b1, W2)


def _tc_post(z2, y2, dinv8, b2, batchc, Wl, bl):
    def body(z_ref, y_ref, dinv_ref, b_ref, batch_ref, wl_ref, bl_ref, o_ref,
             sum_s, cnt_s):
        i = pl.program_id(0)

        @pl.when(i == 0)
        def _():
            sum_s[...] = jnp.zeros((NUM_GRAPHS, FEATS), jnp.float32)
            cnt_s[...] = jnp.zeros((NUM_GRAPHS, FEATS), jnp.float32)

        dinv = _dinv_block(dinv_ref)
        z = z_ref[0] + z_ref[1] + y_ref[...]
        h = jnp.maximum(z * dinv + b_ref[...], 0.0)
        gids = lax.broadcasted_iota(jnp.int32, (NUM_GRAPHS, TCB), 0)
        onehot = (gids == batch_ref[...].reshape(1, TCB)).astype(jnp.float32)
        sum_s[...] += lax.dot_general(onehot, h, (((1,), (0,)), ((), ())),
                                      precision=_HIGH)
        cnt_s[...] += jnp.broadcast_to(
            jnp.sum(onehot, axis=1, keepdims=True), (NUM_GRAPHS, FEATS))

        @pl.when(i == TCG - 1)
        def _():
            pooled = sum_s[...] / jnp.maximum(cnt_s[...], 1.0)
            o_ref[...] = lax.dot_general(
                pooled, wl_ref[...], (((1,), (0,)), ((), ())),
                precision=_HIGH) + bl_ref[...]

    return pl.pallas_call(
        body,
        grid=(TCG,),
        in_specs=[_zrows_spec(), _rows_spec(), _rows_spec(8),
                  _full((1, FEATS)),
                  pl.BlockSpec((1, 1, TCB), lambda i: (i, 0, 0)),
                  _full((FEATS, NUM_CLASSES)), _full((1, NUM_CLASSES))],
        out_specs=pl.BlockSpec((NUM_GRAPHS, NUM_CLASSES), lambda i: (0, 0)),
        out_shape=jax.ShapeDtypeStruct((NUM_GRAPHS, NUM_CLASSES), jnp.float32),
        scratch_shapes=[pltpu.VMEM((NUM_GRAPHS, FEATS), jnp.float32),
                        pltpu.VMEM((NUM_GRAPHS, FEATS), jnp.float32)],
    )(z2, y2, dinv8, b2, batchc, Wl, bl)


def kernel(x, edge_index, batch, W1, b1, W2, b2, Wl, bl):
    src = edge_index[0].astype(jnp.int32)
    dst = edge_index[1].astype(jnp.int32)
    batchc = batch.astype(jnp.int32).reshape(TCG, 1, TCB)

    deg2d = _deg_sc(dst)
    dinv8 = _tc_dinv(deg2d)
    y1 = _tc_pre(x, W1, dinv8)
    z1 = _spmm_sc(y1, src, dst)
    y2 = _tc_mid(z1, y1, dinv8, b1.reshape(1, FEATS), W2)
    z2 = _spmm_sc(y2, src, dst)
    return _tc_post(z2, y2, dinv8, b2.reshape(1, FEATS), batchc,
                    Wl, bl.reshape(1, NUM_CLASSES))


# speedup vs baseline: 22.7860x; 1.0143x over previous
import functools

import jax
import jax.numpy as jnp
from jax import lax
from jax.experimental import pallas as pl
from jax.experimental.pallas import tpu as pltpu
from jax.experimental.pallas import tpu_sc as plsc

N_NODES = 10000
N_EDGES = 320000
FEATS = 128
NUM_GRAPHS = 64
NUM_CLASSES = 10

NC = 2
NS = 16

RCH = 624
TAIL0 = RCH * NS
TAIL = N_NODES - TAIL0
ZT = 156

W_EDGE = 40
KDEPTH = 5

TCB = 2000
TCG = N_NODES // TCB

_HIGH = jax.lax.Precision.HIGHEST


def _mesh():
    return plsc.VectorSubcoreMesh(core_axis_name="c", subcore_axis_name="s")


def _fill(tile, rows, width, value):
    v16 = jnp.full((16,), value, jnp.float32)

    @pl.loop(0, rows)
    def _(i):
        @pl.loop(0, width, step=16)
        def _(j):
            tile[i, pl.ds(j, 16)] = v16


def _clear_and_readout(sid, acc_sp, ztile, out_ref, phase):
    r0 = sid * RCH
    if phase == 0:
        @pl.loop(0, RCH // ZT)
        def _(i):
            pltpu.sync_copy(ztile, acc_sp.at[pl.ds(r0 + i * ZT, ZT)])

        @pl.when(sid == 0)
        def _():
            pltpu.sync_copy(ztile.at[pl.ds(0, TAIL)], acc_sp.at[pl.ds(TAIL0, TAIL)])
    else:
        pltpu.sync_copy(acc_sp.at[pl.ds(r0, RCH)], out_ref.at[pl.ds(r0, RCH)])

        @pl.when(sid == 0)
        def _():
            pltpu.sync_copy(acc_sp.at[pl.ds(TAIL0, TAIL)],
                            out_ref.at[pl.ds(TAIL0, TAIL)])


def _deg_sc(dst):
    @functools.partial(
        pl.kernel,
        out_type=jax.ShapeDtypeStruct((NC, N_NODES, FEATS), jnp.float32),
        mesh=_mesh(),
        scratch_types=(
            [pltpu.VMEM((W_EDGE,), jnp.int32)] * KDEPTH
            + [pltpu.VMEM((W_EDGE, FEATS), jnp.float32),
               pltpu.VMEM((ZT, FEATS), jnp.float32),
               pltpu.VMEM_SHARED((N_NODES, FEATS), jnp.float32),
               pltpu.SemaphoreType.DMA, pltpu.SemaphoreType.DMA]
        ),
    )
    def k(dst_hbm, out_hbm, *scr):
        didx = scr[:KDEPTH]
        ones_v, ztile, deg_sp, sem_i, sem_s = scr[KDEPTH:]
        cid = lax.axis_index("c")
        sid = lax.axis_index("s")
        _fill(ones_v, W_EDGE, FEATS, 1.0)
        _fill(ztile, ZT, FEATS, 0.0)
        _clear_and_readout(sid, deg_sp, ztile, None, 0)
        plsc.subcore_barrier()

        e0 = (cid * NS + sid) * (N_EDGES // (NC * NS))

        @pl.loop(0, (N_EDGES // (NC * NS)) // (W_EDGE * KDEPTH))
        def _(g):
            base = e0 + g * (W_EDGE * KDEPTH)

            @pl.when(g > 0)
            def _():
                for j in range(KDEPTH):
                    pltpu.make_async_copy(ones_v, deg_sp.at[didx[j]],
                                          sem_s).wait()

            for j in range(KDEPTH):
                pltpu.async_copy(dst_hbm.at[pl.ds(base + j * W_EDGE, W_EDGE)],
                                 didx[j], sem_i)
            for j in range(KDEPTH):
                pltpu.make_async_copy(
                    dst_hbm.at[pl.ds(base + j * W_EDGE, W_EDGE)], didx[j],
                    sem_i).wait()
            for j in range(KDEPTH):
                pltpu.async_copy(ones_v, deg_sp.at[didx[j]], sem_s, add=True)

        for j in range(KDEPTH):
            pltpu.make_async_copy(ones_v, deg_sp.at[didx[j]], sem_s).wait()

        plsc.subcore_barrier()
        _clear_and_readout(sid, deg_sp, ztile, out_hbm.at[cid], 1)

    return k(dst)


def _spmm_sc(y, src, dst):
    eps = N_EDGES // (NC * NS)

    @functools.partial(
        pl.kernel,
        out_type=jax.ShapeDtypeStruct((NC, N_NODES, FEATS), jnp.float32),
        mesh=_mesh(),
        scratch_types=(
            [pltpu.VMEM((W_EDGE,), jnp.int32)] * (2 * KDEPTH)
            + [pltpu.VMEM((W_EDGE, FEATS), jnp.float32)] * KDEPTH
            + [pltpu.VMEM((ZT, FEATS), jnp.float32),
               pltpu.VMEM_SHARED((N_NODES, FEATS), jnp.float32),
               pltpu.SemaphoreType.DMA, pltpu.SemaphoreType.DMA,
               pltpu.SemaphoreType.DMA]
        ),
    )
    def k(y_hbm, src_hbm, dst_hbm, out_hbm, *scr):
        sidx = scr[:KDEPTH]
        didx = scr[KDEPTH:2 * KDEPTH]
        rows = scr[2 * KDEPTH:3 * KDEPTH]
        ztile, z_sp, sem_i, sem_g, sem_s = scr[3 * KDEPTH:]
        cid = lax.axis_index("c")
        sid = lax.axis_index("s")

        _fill(ztile, ZT, FEATS, 0.0)
        _clear_and_readout(sid, z_sp, ztile, None, 0)
        plsc.subcore_barrier()

        e0 = (cid * NS + sid) * eps

        @pl.loop(0, eps // (W_EDGE * KDEPTH))
        def _(g):
            base = e0 + g * (W_EDGE * KDEPTH)

            @pl.when(g > 0)
            def _():
                for j in range(KDEPTH):
                    pltpu.make_async_copy(rows[j], z_sp.at[didx[j]],
                                          sem_s).wait()

            for j in range(KDEPTH):
                pltpu.async_copy(src_hbm.at[pl.ds(base + j * W_EDGE, W_EDGE)],
                                 sidx[j], sem_i)
                pltpu.async_copy(dst_hbm.at[pl.ds(base + j * W_EDGE, W_EDGE)],
                                 didx[j], sem_i)
            for j in range(KDEPTH):
                pltpu.make_async_copy(
                    src_hbm.at[pl.ds(base + j * W_EDGE, W_EDGE)], sidx[j],
                    sem_i).wait()
                pltpu.make_async_copy(
                    dst_hbm.at[pl.ds(base + j * W_EDGE, W_EDGE)], didx[j],
                    sem_i).wait()
            for j in range(KDEPTH):
                pltpu.async_copy(y_hbm.at[sidx[j]], rows[j], sem_g)
            for j in range(KDEPTH):
                pltpu.make_async_copy(y_hbm.at[sidx[j]], rows[j], sem_g).wait()
                pltpu.async_copy(rows[j], z_sp.at[didx[j]], sem_s, add=True)

        for j in range(KDEPTH):
            pltpu.make_async_copy(rows[j], z_sp.at[didx[j]], sem_s).wait()

        plsc.subcore_barrier()
        _clear_and_readout(sid, z_sp, ztile, out_hbm.at[cid], 1)

    return k(y, src, dst)


def _tc_mm(x, W1):
    def body(x_ref, w_ref, o_ref):
        o_ref[...] = lax.dot_general(x_ref[...], w_ref[...],
                                     (((1,), (0,)), ((), ())), precision=_HIGH)

    return pl.pallas_call(
        body,
        grid=(TCG,),
        in_specs=[_rows_spec(), _full((FEATS, FEATS))],
        out_specs=_rows_spec(),
        out_shape=jax.ShapeDtypeStruct((N_NODES, FEATS), jnp.float32),
    )(x, W1)


def _tc_scale(xw, deg2d):
    def body(xw_ref, deg_ref, y_ref, dinv_ref):
        d = deg_ref[...]
        deg = (d[0, :, 0] + d[1, :, 0]).reshape(TCB, 1) + 1.0
        dinv = 1.0 / jnp.sqrt(deg)
        y_ref[...] = xw_ref[...] * dinv
        dinv_ref[...] = jnp.broadcast_to(dinv, (TCB, 8))

    return pl.pallas_call(
        body,
        grid=(TCG,),
        in_specs=[_rows_spec(), _zrows_spec()],
        out_specs=[_rows_spec(), _rows_spec(8)],
        out_shape=[jax.ShapeDtypeStruct((N_NODES, FEATS), jnp.float32),
                   jax.ShapeDtypeStruct((N_NODES, 8), jnp.float32)],
    )(xw, deg2d)


def _dinv_block(dinv_ref):
    return dinv_ref[...][:, 0:1]


def _rows_spec(width=FEATS):
    return pl.BlockSpec((TCB, width), lambda i: (i, 0))


def _zrows_spec(width=FEATS):
    return pl.BlockSpec((NC, TCB, width), lambda i: (0, i, 0))


def _full(shape):
    n = len(shape)
    return pl.BlockSpec(shape, lambda i, _n=n: (0,) * _n)


def _tc_mid(z1, y1, dinv8, b1, W2):
    def body(z_ref, y_ref, dinv_ref, b_ref, w_ref, o_ref):
        dinv = _dinv_block(dinv_ref)
        z = z_ref[0] + z_ref[1] + y_ref[...]
        h = jnp.maximum(z * dinv + b_ref[...], 0.0)
        o_ref[...] = lax.dot_general(h, w_ref[...], (((1,), (0,)), ((), ())),
                                     precision=_HIGH) * dinv

    return pl.pallas_call(
        body,
        grid=(TCG,),
        in_specs=[_zrows_spec(), _rows_spec(), _rows_spec(8),
                  _full((1, FEATS)), _full((FEATS, FEATS))],
        out_specs=_rows_spec(),
        out_shape=jax.ShapeDtypeStruct((N_NODES, FEATS), jnp.float32),
    )(z1, y1, dinv8, b1, W2)


def _tc_post(z2, y2, dinv8, b2, batchc, Wl, bl):
    def body(z_ref, y_ref, dinv_ref, b_ref, batch_ref, wl_ref, bl_ref, o_ref,
             sum_s, cnt_s):
        i = pl.program_id(0)

        @pl.when(i == 0)
        def _():
            sum_s[...] = jnp.zeros((NUM_GRAPHS, FEATS), jnp.float32)
            cnt_s[...] = jnp.zeros((NUM_GRAPHS, FEATS), jnp.float32)

        dinv = _dinv_block(dinv_ref)
        z = z_ref[0] + z_ref[1] + y_ref[...]
        h = jnp.maximum(z * dinv + b_ref[...], 0.0)
        gids = lax.broadcasted_iota(jnp.int32, (NUM_GRAPHS, TCB), 0)
        onehot = (gids == batch_ref[...].reshape(1, TCB)).astype(jnp.float32)
        sum_s[...] += lax.dot_general(onehot, h, (((1,), (0,)), ((), ())),
                                      precision=_HIGH)
        cnt_s[...] += jnp.broadcast_to(
            jnp.sum(onehot, axis=1, keepdims=True), (NUM_GRAPHS, FEATS))

        @pl.when(i == TCG - 1)
        def _():
            pooled = sum_s[...] / jnp.maximum(cnt_s[...], 1.0)
            o_ref[...] = lax.dot_general(
                pooled, wl_ref[...], (((1,), (0,)), ((), ())),
                precision=_HIGH) + bl_ref[...]

    return pl.pallas_call(
        body,
        grid=(TCG,),
        in_specs=[_zrows_spec(), _rows_spec(), _rows_spec(8),
                  _full((1, FEATS)),
                  pl.BlockSpec((1, 1, TCB), lambda i: (i, 0, 0)),
                  _full((FEATS, NUM_CLASSES)), _full((1, NUM_CLASSES))],
        out_specs=pl.BlockSpec((NUM_GRAPHS, NUM_CLASSES), lambda i: (0, 0)),
        out_shape=jax.ShapeDtypeStruct((NUM_GRAPHS, NUM_CLASSES), jnp.float32),
        scratch_shapes=[pltpu.VMEM((NUM_GRAPHS, FEATS), jnp.float32),
                        pltpu.VMEM((NUM_GRAPHS, FEATS), jnp.float32)],
    )(z2, y2, dinv8, b2, batchc, Wl, bl)


def kernel(x, edge_index, batch, W1, b1, W2, b2, Wl, bl):
    src = edge_index[0].astype(jnp.int32)
    dst = edge_index[1].astype(jnp.int32)
    batchc = batch.astype(jnp.int32).reshape(TCG, 1, TCB)

    deg2d = _deg_sc(dst)
    xw = _tc_mm(x, W1)
    y1, dinv8 = _tc_scale(xw, deg2d)
    z1 = _spmm_sc(y1, src, dst)
    y2 = _tc_mid(z1, y1, dinv8, b1.reshape(1, FEATS), W2)
    z2 = _spmm_sc(y2, src, dst)
    return _tc_post(z2, y2, dinv8, b2.reshape(1, FEATS), batchc,
                    Wl, bl.reshape(1, NUM_CLASSES))


# speedup vs baseline: 25.7260x; 1.1290x over previous
import functools

import jax
import jax.numpy as jnp
from jax import lax
from jax.experimental import pallas as pl
from jax.experimental.pallas import tpu as pltpu
from jax.experimental.pallas import tpu_sc as plsc

N_NODES = 10000
N_EDGES = 320000
FEATS = 128
NUM_GRAPHS = 64
NUM_CLASSES = 10

NC = 2
NS = 16

RCH = 624
TAIL0 = RCH * NS
TAIL = N_NODES - TAIL0
ZT = 156

W_EDGE = 40
KDEPTH = 5

TCB = 2000
TCG = N_NODES // TCB

_HIGH = jax.lax.Precision.HIGHEST


def _mesh():
    return plsc.VectorSubcoreMesh(core_axis_name="c", subcore_axis_name="s")


def _fill1d(buf, n, value):
    v16 = jnp.full((16,), value, jnp.float32)

    @pl.loop(0, n, step=16)
    def _(i):
        buf[pl.ds(i, 16)] = v16


def _fill(tile, rows, width, value):
    v16 = jnp.full((16,), value, jnp.float32)

    @pl.loop(0, rows)
    def _(i):
        @pl.loop(0, width, step=16)
        def _(j):
            tile[i, pl.ds(j, 16)] = v16


def _clear_and_readout(sid, acc_sp, ztile, out_ref, phase):
    r0 = sid * RCH
    if phase == 0:
        @pl.loop(0, RCH // ZT)
        def _(i):
            pltpu.sync_copy(ztile, acc_sp.at[pl.ds(r0 + i * ZT, ZT)])

        @pl.when(sid == 0)
        def _():
            pltpu.sync_copy(ztile.at[pl.ds(0, TAIL)], acc_sp.at[pl.ds(TAIL0, TAIL)])
    else:
        pltpu.sync_copy(acc_sp.at[pl.ds(r0, RCH)], out_ref.at[pl.ds(r0, RCH)])

        @pl.when(sid == 0)
        def _():
            pltpu.sync_copy(acc_sp.at[pl.ds(TAIL0, TAIL)],
                            out_ref.at[pl.ds(TAIL0, TAIL)])


def _deg_sc(dst):
    @functools.partial(
        pl.kernel,
        out_type=jax.ShapeDtypeStruct((NC, N_NODES), jnp.float32),
        mesh=_mesh(),
        scratch_types=(
            [pltpu.VMEM((W_EDGE,), jnp.int32)] * KDEPTH
            + [pltpu.VMEM((W_EDGE,), jnp.float32),
               pltpu.VMEM((N_NODES,), jnp.float32),
               pltpu.VMEM_SHARED((N_NODES,), jnp.float32),
               pltpu.SemaphoreType.DMA, pltpu.SemaphoreType.DMA]
        ),
    )
    def k(dst_hbm, out_hbm, *scr):
        didx = scr[:KDEPTH]
        ones_v, ztile, deg_sp, sem_i, sem_s = scr[KDEPTH:]
        cid = lax.axis_index("c")
        sid = lax.axis_index("s")
        _fill1d(ones_v, W_EDGE, 1.0)

        @pl.when(sid == 0)
        def _():
            _fill1d(ztile, N_NODES, 0.0)
            pltpu.sync_copy(ztile, deg_sp)

        plsc.subcore_barrier()

        e0 = (cid * NS + sid) * (N_EDGES // (NC * NS))

        @pl.loop(0, (N_EDGES // (NC * NS)) // (W_EDGE * KDEPTH))
        def _(g):
            base = e0 + g * (W_EDGE * KDEPTH)

            @pl.when(g > 0)
            def _():
                for j in range(KDEPTH):
                    pltpu.make_async_copy(ones_v, deg_sp.at[didx[j]],
                                          sem_s).wait()

            for j in range(KDEPTH):
                pltpu.async_copy(dst_hbm.at[pl.ds(base + j * W_EDGE, W_EDGE)],
                                 didx[j], sem_i)
            for j in range(KDEPTH):
                pltpu.make_async_copy(
                    dst_hbm.at[pl.ds(base + j * W_EDGE, W_EDGE)], didx[j],
                    sem_i).wait()
            for j in range(KDEPTH):
                pltpu.async_copy(ones_v, deg_sp.at[didx[j]], sem_s, add=True)

        for j in range(KDEPTH):
            pltpu.make_async_copy(ones_v, deg_sp.at[didx[j]], sem_s).wait()

        plsc.subcore_barrier()

        @pl.when(sid == 0)
        def _():
            pltpu.sync_copy(deg_sp, out_hbm.at[cid])

    return k(dst)


def _spmm_sc(y, src, dst):
    eps = N_EDGES // (NC * NS)

    @functools.partial(
        pl.kernel,
        out_type=jax.ShapeDtypeStruct((NC, N_NODES, FEATS), jnp.float32),
        mesh=_mesh(),
        scratch_types=(
            [pltpu.VMEM((W_EDGE,), jnp.int32)] * (2 * KDEPTH)
            + [pltpu.VMEM((W_EDGE, FEATS), jnp.float32)] * KDEPTH
            + [pltpu.VMEM((ZT, FEATS), jnp.float32),
               pltpu.VMEM_SHARED((N_NODES, FEATS), jnp.float32),
               pltpu.SemaphoreType.DMA, pltpu.SemaphoreType.DMA,
               pltpu.SemaphoreType.DMA]
        ),
    )
    def k(y_hbm, src_hbm, dst_hbm, out_hbm, *scr):
        sidx = scr[:KDEPTH]
        didx = scr[KDEPTH:2 * KDEPTH]
        rows = scr[2 * KDEPTH:3 * KDEPTH]
        ztile, z_sp, sem_i, sem_g, sem_s = scr[3 * KDEPTH:]
        cid = lax.axis_index("c")
        sid = lax.axis_index("s")

        _fill(ztile, ZT, FEATS, 0.0)
        _clear_and_readout(sid, z_sp, ztile, None, 0)
        plsc.subcore_barrier()

        e0 = (cid * NS + sid) * eps

        @pl.loop(0, eps // (W_EDGE * KDEPTH))
        def _(g):
            base = e0 + g * (W_EDGE * KDEPTH)

            @pl.when(g > 0)
            def _():
                for j in range(KDEPTH):
                    pltpu.make_async_copy(rows[j], z_sp.at[didx[j]],
                                          sem_s).wait()

            for j in range(KDEPTH):
                pltpu.async_copy(src_hbm.at[pl.ds(base + j * W_EDGE, W_EDGE)],
                                 sidx[j], sem_i)
                pltpu.async_copy(dst_hbm.at[pl.ds(base + j * W_EDGE, W_EDGE)],
                                 didx[j], sem_i)
            for j in range(KDEPTH):
                pltpu.make_async_copy(
                    src_hbm.at[pl.ds(base + j * W_EDGE, W_EDGE)], sidx[j],
                    sem_i).wait()
                pltpu.make_async_copy(
                    dst_hbm.at[pl.ds(base + j * W_EDGE, W_EDGE)], didx[j],
                    sem_i).wait()
            for j in range(KDEPTH):
                pltpu.async_copy(y_hbm.at[sidx[j]], rows[j], sem_g)
            for j in range(KDEPTH):
                pltpu.make_async_copy(y_hbm.at[sidx[j]], rows[j], sem_g).wait()
                pltpu.async_copy(rows[j], z_sp.at[didx[j]], sem_s, add=True)

        for j in range(KDEPTH):
            pltpu.make_async_copy(rows[j], z_sp.at[didx[j]], sem_s).wait()

        plsc.subcore_barrier()
        _clear_and_readout(sid, z_sp, ztile, out_hbm.at[cid], 1)

    return k(y, src, dst)


def _tc_mm(x, W1):
    def body(x_ref, w_ref, o_ref):
        o_ref[...] = lax.dot_general(x_ref[...], w_ref[...],
                                     (((1,), (0,)), ((), ())), precision=_HIGH)

    return pl.pallas_call(
        body,
        grid=(TCG,),
        in_specs=[_rows_spec(), _full((FEATS, FEATS))],
        out_specs=_rows_spec(),
        out_shape=jax.ShapeDtypeStruct((N_NODES, FEATS), jnp.float32),
    )(x, W1)


def _tc_scale(xw, deg2d):
    def body(xw_ref, deg_ref, y_ref, dinv_ref):
        d = deg_ref[...]
        deg = (d[0, 0] + d[0, 1]).reshape(TCB, 1) + 1.0
        dinv = 1.0 / jnp.sqrt(deg)
        y_ref[...] = xw_ref[...] * dinv
        dinv_ref[...] = jnp.broadcast_to(dinv, (TCB, 8))

    return pl.pallas_call(
        body,
        grid=(TCG,),
        in_specs=[_rows_spec(),
                  pl.BlockSpec((1, NC, TCB), lambda i: (i, 0, 0))],
        out_specs=[_rows_spec(), _rows_spec(8)],
        out_shape=[jax.ShapeDtypeStruct((N_NODES, FEATS), jnp.float32),
                   jax.ShapeDtypeStruct((N_NODES, 8), jnp.float32)],
    )(xw, deg2d)


def _dinv_block(dinv_ref):
    return dinv_ref[...][:, 0:1]


def _rows_spec(width=FEATS):
    return pl.BlockSpec((TCB, width), lambda i: (i, 0))


def _zrows_spec(width=FEATS):
    return pl.BlockSpec((NC, TCB, width), lambda i: (0, i, 0))


def _full(shape):
    n = len(shape)
    return pl.BlockSpec(shape, lambda i, _n=n: (0,) * _n)


def _tc_mid(z1, y1, dinv8, b1, W2):
    def body(z_ref, y_ref, dinv_ref, b_ref, w_ref, o_ref):
        dinv = _dinv_block(dinv_ref)
        z = z_ref[0] + z_ref[1] + y_ref[...]
        h = jnp.maximum(z * dinv + b_ref[...], 0.0)
        o_ref[...] = lax.dot_general(h, w_ref[...], (((1,), (0,)), ((), ())),
                                     precision=_HIGH) * dinv

    return pl.pallas_call(
        body,
        grid=(TCG,),
        in_specs=[_zrows_spec(), _rows_spec(), _rows_spec(8),
                  _full((1, FEATS)), _full((FEATS, FEATS))],
        out_specs=_rows_spec(),
        out_shape=jax.ShapeDtypeStruct((N_NODES, FEATS), jnp.float32),
    )(z1, y1, dinv8, b1, W2)


def _tc_post(z2, y2, dinv8, b2, batchc, Wl, bl):
    def body(z_ref, y_ref, dinv_ref, b_ref, batch_ref, wl_ref, bl_ref, o_ref,
             sum_s, cnt_s):
        i = pl.program_id(0)

        @pl.when(i == 0)
        def _():
            sum_s[...] = jnp.zeros((NUM_GRAPHS, FEATS), jnp.float32)
            cnt_s[...] = jnp.zeros((NUM_GRAPHS, FEATS), jnp.float32)

        dinv = _dinv_block(dinv_ref)
        z = z_ref[0] + z_ref[1] + y_ref[...]
        h = jnp.maximum(z * dinv + b_ref[...], 0.0)
        gids = lax.broadcasted_iota(jnp.int32, (NUM_GRAPHS, TCB), 0)
        onehot = (gids == batch_ref[...].reshape(1, TCB)).astype(jnp.float32)
        sum_s[...] += lax.dot_general(onehot, h, (((1,), (0,)), ((), ())),
                                      precision=_HIGH)
        cnt_s[...] += jnp.broadcast_to(
            jnp.sum(onehot, axis=1, keepdims=True), (NUM_GRAPHS, FEATS))

        @pl.when(i == TCG - 1)
        def _():
            pooled = sum_s[...] / jnp.maximum(cnt_s[...], 1.0)
            o_ref[...] = lax.dot_general(
                pooled, wl_ref[...], (((1,), (0,)), ((), ())),
                precision=_HIGH) + bl_ref[...]

    return pl.pallas_call(
        body,
        grid=(TCG,),
        in_specs=[_zrows_spec(), _rows_spec(), _rows_spec(8),
                  _full((1, FEATS)),
                  pl.BlockSpec((1, 1, TCB), lambda i: (i, 0, 0)),
                  _full((FEATS, NUM_CLASSES)), _full((1, NUM_CLASSES))],
        out_specs=pl.BlockSpec((NUM_GRAPHS, NUM_CLASSES), lambda i: (0, 0)),
        out_shape=jax.ShapeDtypeStruct((NUM_GRAPHS, NUM_CLASSES), jnp.float32),
        scratch_shapes=[pltpu.VMEM((NUM_GRAPHS, FEATS), jnp.float32),
                        pltpu.VMEM((NUM_GRAPHS, FEATS), jnp.float32)],
    )(z2, y2, dinv8, b2, batchc, Wl, bl)


def kernel(x, edge_index, batch, W1, b1, W2, b2, Wl, bl):
    src = edge_index[0].astype(jnp.int32)
    dst = edge_index[1].astype(jnp.int32)
    batchc = batch.astype(jnp.int32).reshape(TCG, 1, TCB)

    deg2d = jnp.transpose(_deg_sc(dst).reshape(NC, TCG, TCB), (1, 0, 2))
    xw = _tc_mm(x, W1)
    y1, dinv8 = _tc_scale(xw, deg2d)
    z1 = _spmm_sc(y1, src, dst)
    y2 = _tc_mid(z1, y1, dinv8, b1.reshape(1, FEATS), W2)
    z2 = _spmm_sc(y2, src, dst)
    return _tc_post(z2, y2, dinv8, b2.reshape(1, FEATS), batchc,
                    Wl, bl.reshape(1, NUM_CLASSES))


# speedup vs baseline: 28.9474x; 1.1252x over previous
import functools

import jax
import jax.numpy as jnp
from jax import lax
from jax.experimental import pallas as pl
from jax.experimental.pallas import tpu as pltpu
from jax.experimental.pallas import tpu_sc as plsc

N_NODES = 10000
N_EDGES = 320000
FEATS = 128
NUM_GRAPHS = 64
NUM_CLASSES = 10

NC = 2
NS = 16

RCH = 624
TAIL0 = RCH * NS
TAIL = N_NODES - TAIL0
ZT = 156

W_EDGE = 40
KDEPTH = 5

TCB = 2000
TCG = N_NODES // TCB

_HIGH = jax.lax.Precision.HIGHEST


def _mesh():
    return plsc.VectorSubcoreMesh(core_axis_name="c", subcore_axis_name="s")


def _fill1d(buf, n, value):
    v16 = jnp.full((16,), value, jnp.float32)

    @pl.loop(0, n, step=16)
    def _(i):
        buf[pl.ds(i, 16)] = v16


def _fill(tile, rows, width, value):
    v16 = jnp.full((16,), value, jnp.float32)

    @pl.loop(0, rows)
    def _(i):
        @pl.loop(0, width, step=16)
        def _(j):
            tile[i, pl.ds(j, 16)] = v16


def _clear_and_readout(sid, acc_sp, ztile, out_ref, phase):
    r0 = sid * RCH
    if phase == 0:
        @pl.loop(0, RCH // ZT)
        def _(i):
            pltpu.sync_copy(ztile, acc_sp.at[pl.ds(r0 + i * ZT, ZT)])

        @pl.when(sid == 0)
        def _():
            pltpu.sync_copy(ztile.at[pl.ds(0, TAIL)], acc_sp.at[pl.ds(TAIL0, TAIL)])
    else:
        pltpu.sync_copy(acc_sp.at[pl.ds(r0, RCH)], out_ref.at[pl.ds(r0, RCH)])

        @pl.when(sid == 0)
        def _():
            pltpu.sync_copy(acc_sp.at[pl.ds(TAIL0, TAIL)],
                            out_ref.at[pl.ds(TAIL0, TAIL)])


def _deg_sc(dst):
    @functools.partial(
        pl.kernel,
        out_type=jax.ShapeDtypeStruct((NC, N_NODES), jnp.float32),
        mesh=_mesh(),
        scratch_types=(
            [pltpu.VMEM((W_EDGE,), jnp.int32)] * KDEPTH
            + [pltpu.VMEM((W_EDGE,), jnp.float32),
               pltpu.VMEM((N_NODES,), jnp.float32),
               pltpu.VMEM_SHARED((N_NODES,), jnp.float32),
               pltpu.SemaphoreType.DMA, pltpu.SemaphoreType.DMA]
        ),
    )
    def k(dst_hbm, out_hbm, *scr):
        didx = scr[:KDEPTH]
        ones_v, ztile, deg_sp, sem_i, sem_s = scr[KDEPTH:]
        cid = lax.axis_index("c")
        sid = lax.axis_index("s")
        _fill1d(ones_v, W_EDGE, 1.0)

        @pl.when(sid == 0)
        def _():
            _fill1d(ztile, N_NODES, 0.0)
            pltpu.sync_copy(ztile, deg_sp)

        plsc.subcore_barrier()

        e0 = (cid * NS + sid) * (N_EDGES // (NC * NS))

        @pl.loop(0, (N_EDGES // (NC * NS)) // (W_EDGE * KDEPTH))
        def _(g):
            base = e0 + g * (W_EDGE * KDEPTH)

            @pl.when(g > 0)
            def _():
                for j in range(KDEPTH):
                    pltpu.make_async_copy(ones_v, deg_sp.at[didx[j]],
                                          sem_s).wait()

            for j in range(KDEPTH):
                pltpu.async_copy(dst_hbm.at[pl.ds(base + j * W_EDGE, W_EDGE)],
                                 didx[j], sem_i)
            for j in range(KDEPTH):
                pltpu.make_async_copy(
                    dst_hbm.at[pl.ds(base + j * W_EDGE, W_EDGE)], didx[j],
                    sem_i).wait()
            for j in range(KDEPTH):
                pltpu.async_copy(ones_v, deg_sp.at[didx[j]], sem_s, add=True)

        for j in range(KDEPTH):
            pltpu.make_async_copy(ones_v, deg_sp.at[didx[j]], sem_s).wait()

        plsc.subcore_barrier()

        @pl.when(sid == 0)
        def _():
            pltpu.sync_copy(deg_sp, out_hbm.at[cid])

    return k(dst)


def _spmm_sc(y, src, dst):
    eps = N_EDGES // (NC * NS)

    gsz = W_EDGE * KDEPTH
    ngrp = eps // gsz

    @functools.partial(
        pl.kernel,
        out_type=jax.ShapeDtypeStruct((NC, N_NODES, FEATS), jnp.float32),
        mesh=_mesh(),
        scratch_types=(
            [pltpu.VMEM((W_EDGE,), jnp.int32)] * (4 * KDEPTH)
            + [pltpu.VMEM((W_EDGE, FEATS), jnp.float32)] * KDEPTH
            + [pltpu.VMEM((ZT, FEATS), jnp.float32),
               pltpu.VMEM_SHARED((N_NODES, FEATS), jnp.float32),
               pltpu.SemaphoreType.DMA, pltpu.SemaphoreType.DMA,
               pltpu.SemaphoreType.DMA]
        ),
    )
    def k(y_hbm, src_hbm, dst_hbm, out_hbm, *scr):
        sidx = (scr[:KDEPTH], scr[KDEPTH:2 * KDEPTH])
        didx = (scr[2 * KDEPTH:3 * KDEPTH], scr[3 * KDEPTH:4 * KDEPTH])
        rows = scr[4 * KDEPTH:5 * KDEPTH]
        ztile, z_sp, sem_i, sem_g, sem_s = scr[5 * KDEPTH:]
        cid = lax.axis_index("c")
        sid = lax.axis_index("s")

        _fill(ztile, ZT, FEATS, 0.0)
        _clear_and_readout(sid, z_sp, ztile, None, 0)
        plsc.subcore_barrier()

        e0 = (cid * NS + sid) * eps

        def fire_idx(g, s):
            base = e0 + g * gsz
            for j in range(KDEPTH):
                pltpu.async_copy(src_hbm.at[pl.ds(base + j * W_EDGE, W_EDGE)],
                                 sidx[s][j], sem_i)
                pltpu.async_copy(dst_hbm.at[pl.ds(base + j * W_EDGE, W_EDGE)],
                                 didx[s][j], sem_i)

        def drain_idx(g, s):
            base = e0 + g * gsz
            for j in range(KDEPTH):
                pltpu.make_async_copy(
                    src_hbm.at[pl.ds(base + j * W_EDGE, W_EDGE)], sidx[s][j],
                    sem_i).wait()
                pltpu.make_async_copy(
                    dst_hbm.at[pl.ds(base + j * W_EDGE, W_EDGE)], didx[s][j],
                    sem_i).wait()

        def drain_scatters(s):
            for j in range(KDEPTH):
                pltpu.make_async_copy(rows[j], z_sp.at[didx[s][j]],
                                      sem_s).wait()

        fire_idx(0, 0)

        @pl.loop(0, ngrp // 2)
        def _(gg):
            for sub in range(2):
                g = gg * 2 + sub
                s = sub

                @pl.when(g > 0)
                def _():
                    drain_scatters(1 - s)

                drain_idx(g, s)
                for j in range(KDEPTH):
                    pltpu.async_copy(y_hbm.at[sidx[s][j]], rows[j], sem_g)

                @pl.when(g + 1 < ngrp)
                def _():
                    fire_idx(g + 1, 1 - s)

                for j in range(KDEPTH):
                    pltpu.make_async_copy(y_hbm.at[sidx[s][j]], rows[j],
                                          sem_g).wait()
                    pltpu.async_copy(rows[j], z_sp.at[didx[s][j]], sem_s,
                                     add=True)

        drain_scatters(1)

        plsc.subcore_barrier()
        _clear_and_readout(sid, z_sp, ztile, out_hbm.at[cid], 1)

    return k(y, src, dst)


def _tc_mm(x, W1):
    def body(x_ref, w_ref, o_ref):
        o_ref[...] = lax.dot_general(x_ref[...], w_ref[...],
                                     (((1,), (0,)), ((), ())), precision=_HIGH)

    return pl.pallas_call(
        body,
        grid=(TCG,),
        in_specs=[_rows_spec(), _full((FEATS, FEATS))],
        out_specs=_rows_spec(),
        out_shape=jax.ShapeDtypeStruct((N_NODES, FEATS), jnp.float32),
    )(x, W1)


def _tc_scale(xw, deg2d):
    def body(xw_ref, deg_ref, y_ref, dinv_ref):
        d = deg_ref[...]
        deg = (d[0, 0] + d[0, 1]).reshape(TCB, 1) + 1.0
        dinv = 1.0 / jnp.sqrt(deg)
        y_ref[...] = xw_ref[...] * dinv
        dinv_ref[...] = jnp.broadcast_to(dinv, (TCB, 8))

    return pl.pallas_call(
        body,
        grid=(TCG,),
        in_specs=[_rows_spec(),
                  pl.BlockSpec((1, NC, TCB), lambda i: (i, 0, 0))],
        out_specs=[_rows_spec(), _rows_spec(8)],
        out_shape=[jax.ShapeDtypeStruct((N_NODES, FEATS), jnp.float32),
                   jax.ShapeDtypeStruct((N_NODES, 8), jnp.float32)],
    )(xw, deg2d)


def _dinv_block(dinv_ref):
    return dinv_ref[...][:, 0:1]


def _rows_spec(width=FEATS):
    return pl.BlockSpec((TCB, width), lambda i: (i, 0))


def _zrows_spec(width=FEATS):
    return pl.BlockSpec((NC, TCB, width), lambda i: (0, i, 0))


def _full(shape):
    n = len(shape)
    return pl.BlockSpec(shape, lambda i, _n=n: (0,) * _n)


def _tc_mid(z1, y1, dinv8, b1, W2):
    def body(z_ref, y_ref, dinv_ref, b_ref, w_ref, o_ref):
        dinv = _dinv_block(dinv_ref)
        z = z_ref[0] + z_ref[1] + y_ref[...]
        h = jnp.maximum(z * dinv + b_ref[...], 0.0)
        o_ref[...] = lax.dot_general(h, w_ref[...], (((1,), (0,)), ((), ())),
                                     precision=_HIGH) * dinv

    return pl.pallas_call(
        body,
        grid=(TCG,),
        in_specs=[_zrows_spec(), _rows_spec(), _rows_spec(8),
                  _full((1, FEATS)), _full((FEATS, FEATS))],
        out_specs=_rows_spec(),
        out_shape=jax.ShapeDtypeStruct((N_NODES, FEATS), jnp.float32),
    )(z1, y1, dinv8, b1, W2)


def _tc_post(z2, y2, dinv8, b2, batchc, Wl, bl):
    def body(z_ref, y_ref, dinv_ref, b_ref, batch_ref, wl_ref, bl_ref, o_ref,
             sum_s, cnt_s):
        i = pl.program_id(0)

        @pl.when(i == 0)
        def _():
            sum_s[...] = jnp.zeros((NUM_GRAPHS, FEATS), jnp.float32)
            cnt_s[...] = jnp.zeros((NUM_GRAPHS, FEATS), jnp.float32)

        dinv = _dinv_block(dinv_ref)
        z = z_ref[0] + z_ref[1] + y_ref[...]
        h = jnp.maximum(z * dinv + b_ref[...], 0.0)
        gids = lax.broadcasted_iota(jnp.int32, (NUM_GRAPHS, TCB), 0)
        onehot = (gids == batch_ref[...].reshape(1, TCB)).astype(jnp.float32)
        sum_s[...] += lax.dot_general(onehot, h, (((1,), (0,)), ((), ())),
                                      precision=_HIGH)
        cnt_s[...] += jnp.broadcast_to(
            jnp.sum(onehot, axis=1, keepdims=True), (NUM_GRAPHS, FEATS))

        @pl.when(i == TCG - 1)
        def _():
            pooled = sum_s[...] / jnp.maximum(cnt_s[...], 1.0)
            o_ref[...] = lax.dot_general(
                pooled, wl_ref[...], (((1,), (0,)), ((), ())),
                precision=_HIGH) + bl_ref[...]

    return pl.pallas_call(
        body,
        grid=(TCG,),
        in_specs=[_zrows_spec(), _rows_spec(), _rows_spec(8),
                  _full((1, FEATS)),
                  pl.BlockSpec((1, 1, TCB), lambda i: (i, 0, 0)),
                  _full((FEATS, NUM_CLASSES)), _full((1, NUM_CLASSES))],
        out_specs=pl.BlockSpec((NUM_GRAPHS, NUM_CLASSES), lambda i: (0, 0)),
        out_shape=jax.ShapeDtypeStruct((NUM_GRAPHS, NUM_CLASSES), jnp.float32),
        scratch_shapes=[pltpu.VMEM((NUM_GRAPHS, FEATS), jnp.float32),
                        pltpu.VMEM((NUM_GRAPHS, FEATS), jnp.float32)],
    )(z2, y2, dinv8, b2, batchc, Wl, bl)


def kernel(x, edge_index, batch, W1, b1, W2, b2, Wl, bl):
    src = edge_index[0].astype(jnp.int32)
    dst = edge_index[1].astype(jnp.int32)
    batchc = batch.astype(jnp.int32).reshape(TCG, 1, TCB)

    deg2d = jnp.transpose(_deg_sc(dst).reshape(NC, TCG, TCB), (1, 0, 2))
    xw = _tc_mm(x, W1)
    y1, dinv8 = _tc_scale(xw, deg2d)
    z1 = _spmm_sc(y1, src, dst)
    y2 = _tc_mid(z1, y1, dinv8, b1.reshape(1, FEATS), W2)
    z2 = _spmm_sc(y2, src, dst)
    return _tc_post(z2, y2, dinv8, b2.reshape(1, FEATS), batchc,
                    Wl, bl.reshape(1, NUM_CLASSES))


# speedup vs baseline: 30.7151x; 1.0611x over previous
import functools

import jax
import jax.numpy as jnp
from jax import lax
from jax.experimental import pallas as pl
from jax.experimental.pallas import tpu as pltpu
from jax.experimental.pallas import tpu_sc as plsc

N_NODES = 10000
N_EDGES = 320000
FEATS = 128
NUM_GRAPHS = 64
NUM_CLASSES = 10

NC = 2
NS = 16

RCH = 624
TAIL0 = RCH * NS
TAIL = N_NODES - TAIL0
ZT = 156

W_EDGE = 40
KDEPTH = 5
KD_DEG = 25

TCB = 2000
TCG = N_NODES // TCB

_HIGH = jax.lax.Precision.HIGHEST


def _mesh():
    return plsc.VectorSubcoreMesh(core_axis_name="c", subcore_axis_name="s")


def _fill1d(buf, n, value):
    v16 = jnp.full((16,), value, jnp.float32)

    @pl.loop(0, n, step=16)
    def _(i):
        buf[pl.ds(i, 16)] = v16


def _fill(tile, rows, width, value):
    v16 = jnp.full((16,), value, jnp.float32)

    @pl.loop(0, rows)
    def _(i):
        @pl.loop(0, width, step=16)
        def _(j):
            tile[i, pl.ds(j, 16)] = v16


def _clear_and_readout(sid, acc_sp, ztile, out_ref, phase):
    r0 = sid * RCH
    if phase == 0:
        @pl.loop(0, RCH // ZT)
        def _(i):
            pltpu.sync_copy(ztile, acc_sp.at[pl.ds(r0 + i * ZT, ZT)])

        @pl.when(sid == 0)
        def _():
            pltpu.sync_copy(ztile.at[pl.ds(0, TAIL)], acc_sp.at[pl.ds(TAIL0, TAIL)])
    else:
        pltpu.sync_copy(acc_sp.at[pl.ds(r0, RCH)], out_ref.at[pl.ds(r0, RCH)])

        @pl.when(sid == 0)
        def _():
            pltpu.sync_copy(acc_sp.at[pl.ds(TAIL0, TAIL)],
                            out_ref.at[pl.ds(TAIL0, TAIL)])


def _deg_sc(dst):
    @functools.partial(
        pl.kernel,
        out_type=jax.ShapeDtypeStruct((NC, N_NODES), jnp.float32),
        mesh=_mesh(),
        scratch_types=(
            [pltpu.VMEM((W_EDGE,), jnp.int32)] * KD_DEG
            + [pltpu.VMEM((W_EDGE,), jnp.float32),
               pltpu.VMEM((N_NODES,), jnp.float32),
               pltpu.VMEM_SHARED((N_NODES,), jnp.float32),
               pltpu.SemaphoreType.DMA, pltpu.SemaphoreType.DMA]
        ),
    )
    def k(dst_hbm, out_hbm, *scr):
        didx = scr[:KD_DEG]
        ones_v, ztile, deg_sp, sem_i, sem_s = scr[KD_DEG:]
        cid = lax.axis_index("c")
        sid = lax.axis_index("s")
        _fill1d(ones_v, W_EDGE, 1.0)

        @pl.when(sid == 0)
        def _():
            _fill1d(ztile, N_NODES, 0.0)
            pltpu.sync_copy(ztile, deg_sp)

        plsc.subcore_barrier()

        e0 = (cid * NS + sid) * (N_EDGES // (NC * NS))

        @pl.loop(0, (N_EDGES // (NC * NS)) // (W_EDGE * KD_DEG))
        def _(g):
            base = e0 + g * (W_EDGE * KD_DEG)

            @pl.when(g > 0)
            def _():
                for j in range(KD_DEG):
                    pltpu.make_async_copy(ones_v, deg_sp.at[didx[j]],
                                          sem_s).wait()

            for j in range(KD_DEG):
                pltpu.async_copy(dst_hbm.at[pl.ds(base + j * W_EDGE, W_EDGE)],
                                 didx[j], sem_i)
            for j in range(KD_DEG):
                pltpu.make_async_copy(
                    dst_hbm.at[pl.ds(base + j * W_EDGE, W_EDGE)], didx[j],
                    sem_i).wait()
            for j in range(KD_DEG):
                pltpu.async_copy(ones_v, deg_sp.at[didx[j]], sem_s, add=True)

        for j in range(KD_DEG):
            pltpu.make_async_copy(ones_v, deg_sp.at[didx[j]], sem_s).wait()

        plsc.subcore_barrier()

        @pl.when(sid == 0)
        def _():
            pltpu.sync_copy(deg_sp, out_hbm.at[cid])

    return k(dst)


def _spmm_sc(y, src, dst):
    eps = N_EDGES // (NC * NS)

    gsz = W_EDGE * KDEPTH
    ngrp = eps // gsz

    @functools.partial(
        pl.kernel,
        out_type=jax.ShapeDtypeStruct((NC, N_NODES, FEATS), jnp.float32),
        mesh=_mesh(),
        scratch_types=(
            [pltpu.VMEM((W_EDGE,), jnp.int32)] * (4 * KDEPTH)
            + [pltpu.VMEM((W_EDGE, FEATS), jnp.float32)] * KDEPTH
            + [pltpu.VMEM((ZT, FEATS), jnp.float32),
               pltpu.VMEM_SHARED((N_NODES, FEATS), jnp.float32),
               pltpu.SemaphoreType.DMA, pltpu.SemaphoreType.DMA,
               pltpu.SemaphoreType.DMA]
        ),
    )
    def k(y_hbm, src_hbm, dst_hbm, out_hbm, *scr):
        sidx = (scr[:KDEPTH], scr[KDEPTH:2 * KDEPTH])
        didx = (scr[2 * KDEPTH:3 * KDEPTH], scr[3 * KDEPTH:4 * KDEPTH])
        rows = scr[4 * KDEPTH:5 * KDEPTH]
        ztile, z_sp, sem_i, sem_g, sem_s = scr[5 * KDEPTH:]
        cid = lax.axis_index("c")
        sid = lax.axis_index("s")

        _fill(ztile, ZT, FEATS, 0.0)
        _clear_and_readout(sid, z_sp, ztile, None, 0)
        plsc.subcore_barrier()

        e0 = (cid * NS + sid) * eps

        def fire_idx(g, s):
            base = e0 + g * gsz
            for j in range(KDEPTH):
                pltpu.async_copy(src_hbm.at[pl.ds(base + j * W_EDGE, W_EDGE)],
                                 sidx[s][j], sem_i)
                pltpu.async_copy(dst_hbm.at[pl.ds(base + j * W_EDGE, W_EDGE)],
                                 didx[s][j], sem_i)

        def drain_idx(g, s):
            base = e0 + g * gsz
            for j in range(KDEPTH):
                pltpu.make_async_copy(
                    src_hbm.at[pl.ds(base + j * W_EDGE, W_EDGE)], sidx[s][j],
                    sem_i).wait()
                pltpu.make_async_copy(
                    dst_hbm.at[pl.ds(base + j * W_EDGE, W_EDGE)], didx[s][j],
                    sem_i).wait()

        def drain_scatters(s):
            for j in range(KDEPTH):
                pltpu.make_async_copy(rows[j], z_sp.at[didx[s][j]],
                                      sem_s).wait()

        fire_idx(0, 0)

        @pl.loop(0, ngrp // 2)
        def _(gg):
            for sub in range(2):
                g = gg * 2 + sub
                s = sub

                @pl.when(g > 0)
                def _():
                    drain_scatters(1 - s)

                drain_idx(g, s)
                for j in range(KDEPTH):
                    pltpu.async_copy(y_hbm.at[sidx[s][j]], rows[j], sem_g)

                @pl.when(g + 1 < ngrp)
                def _():
                    fire_idx(g + 1, 1 - s)

                for j in range(KDEPTH):
                    pltpu.make_async_copy(y_hbm.at[sidx[s][j]], rows[j],
                                          sem_g).wait()
                    pltpu.async_copy(rows[j], z_sp.at[didx[s][j]], sem_s,
                                     add=True)

        drain_scatters(1)

        plsc.subcore_barrier()
        _clear_and_readout(sid, z_sp, ztile, out_hbm.at[cid], 1)

    return k(y, src, dst)


def _tc_mm(x, W1):
    def body(x_ref, w_ref, o_ref):
        o_ref[...] = lax.dot_general(x_ref[...], w_ref[...],
                                     (((1,), (0,)), ((), ())), precision=_HIGH)

    return pl.pallas_call(
        body,
        grid=(TCG,),
        in_specs=[_rows_spec(), _full((FEATS, FEATS))],
        out_specs=_rows_spec(),
        out_shape=jax.ShapeDtypeStruct((N_NODES, FEATS), jnp.float32),
    )(x, W1)


def _tc_scale(xw, deg2d):
    def body(xw_ref, deg_ref, y_ref, dinv_ref):
        d = deg_ref[...]
        deg = (d[0, 0] + d[0, 1]).reshape(TCB, 1) + 1.0
        dinv = 1.0 / jnp.sqrt(deg)
        y_ref[...] = xw_ref[...] * dinv
        dinv_ref[...] = jnp.broadcast_to(dinv, (TCB, 8))

    return pl.pallas_call(
        body,
        grid=(TCG,),
        in_specs=[_rows_spec(),
                  pl.BlockSpec((1, NC, TCB), lambda i: (i, 0, 0))],
        out_specs=[_rows_spec(), _rows_spec(8)],
        out_shape=[jax.ShapeDtypeStruct((N_NODES, FEATS), jnp.float32),
                   jax.ShapeDtypeStruct((N_NODES, 8), jnp.float32)],
    )(xw, deg2d)


def _dinv_block(dinv_ref):
    return dinv_ref[...][:, 0:1]


def _rows_spec(width=FEATS):
    return pl.BlockSpec((TCB, width), lambda i: (i, 0))


def _zrows_spec(width=FEATS):
    return pl.BlockSpec((NC, TCB, width), lambda i: (0, i, 0))


def _full(shape):
    n = len(shape)
    return pl.BlockSpec(shape, lambda i, _n=n: (0,) * _n)


def _tc_mid(z1, y1, dinv8, b1, W2):
    def body(z_ref, y_ref, dinv_ref, b_ref, w_ref, o_ref):
        dinv = _dinv_block(dinv_ref)
        z = z_ref[0] + z_ref[1] + y_ref[...]
        h = jnp.maximum(z * dinv + b_ref[...], 0.0)
        o_ref[...] = lax.dot_general(h, w_ref[...], (((1,), (0,)), ((), ())),
                                     precision=_HIGH) * dinv

    return pl.pallas_call(
        body,
        grid=(TCG,),
        in_specs=[_zrows_spec(), _rows_spec(), _rows_spec(8),
                  _full((1, FEATS)), _full((FEATS, FEATS))],
        out_specs=_rows_spec(),
        out_shape=jax.ShapeDtypeStruct((N_NODES, FEATS), jnp.float32),
    )(z1, y1, dinv8, b1, W2)


def _tc_post(z2, y2, dinv8, b2, batchc, Wl, bl):
    def body(z_ref, y_ref, dinv_ref, b_ref, batch_ref, wl_ref, bl_ref, o_ref,
             sum_s, cnt_s):
        i = pl.program_id(0)

        @pl.when(i == 0)
        def _():
            sum_s[...] = jnp.zeros((NUM_GRAPHS, FEATS), jnp.float32)
            cnt_s[...] = jnp.zeros((NUM_GRAPHS, FEATS), jnp.float32)

        dinv = _dinv_block(dinv_ref)
        z = z_ref[0] + z_ref[1] + y_ref[...]
        h = jnp.maximum(z * dinv + b_ref[...], 0.0)
        gids = lax.broadcasted_iota(jnp.int32, (NUM_GRAPHS, TCB), 0)
        onehot = (gids == batch_ref[...].reshape(1, TCB)).astype(jnp.float32)
        sum_s[...] += lax.dot_general(onehot, h, (((1,), (0,)), ((), ())),
                                      precision=_HIGH)
        cnt_s[...] += jnp.broadcast_to(
            jnp.sum(onehot, axis=1, keepdims=True), (NUM_GRAPHS, FEATS))

        @pl.when(i == TCG - 1)
        def _():
            pooled = sum_s[...] / jnp.maximum(cnt_s[...], 1.0)
            o_ref[...] = lax.dot_general(
                pooled, wl_ref[...], (((1,), (0,)), ((), ())),
                precision=_HIGH) + bl_ref[...]

    return pl.pallas_call(
        body,
        grid=(TCG,),
        in_specs=[_zrows_spec(), _rows_spec(), _rows_spec(8),
                  _full((1, FEATS)),
                  pl.BlockSpec((1, 1, TCB), lambda i: (i, 0, 0)),
                  _full((FEATS, NUM_CLASSES)), _full((1, NUM_CLASSES))],
        out_specs=pl.BlockSpec((NUM_GRAPHS, NUM_CLASSES), lambda i: (0, 0)),
        out_shape=jax.ShapeDtypeStruct((NUM_GRAPHS, NUM_CLASSES), jnp.float32),
        scratch_shapes=[pltpu.VMEM((NUM_GRAPHS, FEATS), jnp.float32),
                        pltpu.VMEM((NUM_GRAPHS, FEATS), jnp.float32)],
    )(z2, y2, dinv8, b2, batchc, Wl, bl)


def kernel(x, edge_index, batch, W1, b1, W2, b2, Wl, bl):
    src = edge_index[0].astype(jnp.int32)
    dst = edge_index[1].astype(jnp.int32)
    batchc = batch.astype(jnp.int32).reshape(TCG, 1, TCB)

    deg2d = jnp.transpose(_deg_sc(dst).reshape(NC, TCG, TCB), (1, 0, 2))
    xw = _tc_mm(x, W1)
    y1, dinv8 = _tc_scale(xw, deg2d)
    z1 = _spmm_sc(y1, src, dst)
    y2 = _tc_mid(z1, y1, dinv8, b1.reshape(1, FEATS), W2)
    z2 = _spmm_sc(y2, src, dst)
    return _tc_post(z2, y2, dinv8, b2.reshape(1, FEATS), batchc,
                    Wl, bl.reshape(1, NUM_CLASSES))
